# baseline (speedup 1.0000x reference)
import jax
import jax.numpy as jnp
from jax.experimental import pallas as pl
from jax.experimental.pallas import tpu as pltpu

VOCAB = 1000
ROWS_PER_BLOCK = 256


def _onehot_block(x_ref, o_ref):
    idx = x_ref[0, 0, :].reshape(ROWS_PER_BLOCK, 1)
    iota = jax.lax.broadcasted_iota(jnp.int32, (ROWS_PER_BLOCK, VOCAB), 1)
    o_ref[...] = (idx == iota).astype(jnp.float32)


def kernel(x):
    B, S = x.shape
    n = B * S
    nblocks = n // ROWS_PER_BLOCK
    xi = x.astype(jnp.int32).reshape(nblocks, 1, ROWS_PER_BLOCK)
    out = pl.pallas_call(
        _onehot_block,
        grid=(nblocks,),
        in_specs=[pl.BlockSpec((1, 1, ROWS_PER_BLOCK), lambda i: (i, 0, 0))],
        out_specs=pl.BlockSpec((ROWS_PER_BLOCK, VOCAB), lambda i: (i, 0)),
        out_shape=jax.ShapeDtypeStruct((n, VOCAB), jnp.float32),
    )(xi)
    return out.reshape(B, S, VOCAB)


# speedup vs baseline: 1.7199x; 1.7199x over previous
import jax
import jax.numpy as jnp
from jax.experimental import pallas as pl
from jax.experimental.pallas import tpu as pltpu

VOCAB = 1000
BATCH_BLOCK = 64


def _onehot_block(x_ref, o_ref):
    bb, s = x_ref.shape
    idx = x_ref[...].reshape(bb, s, 1)
    iota = jax.lax.broadcasted_iota(jnp.int32, (bb, s, VOCAB), 2)
    o_ref[...] = (idx == iota).astype(jnp.float32)


def kernel(x):
    B, S = x.shape
    xi = x.astype(jnp.int32)
    nblocks = B // BATCH_BLOCK
    out = pl.pallas_call(
        _onehot_block,
        grid=(nblocks,),
        in_specs=[pl.BlockSpec((BATCH_BLOCK, S), lambda i: (i, 0))],
        out_specs=pl.BlockSpec((BATCH_BLOCK, S, VOCAB), lambda i: (i, 0, 0)),
        out_shape=jax.ShapeDtypeStruct((B, S, VOCAB), jnp.float32),
    )(xi)
    return out
